# baseline (speedup 1.0000x reference)
import functools

import jax
import jax.numpy as jnp
from jax.experimental import pallas as pl
from jax.experimental.pallas import tpu as pltpu

S = 1024
BLK = 128
NBLK = S // BLK


def _mv(v, W):
    return jax.lax.dot_general(
        v, W, (((1,), (1,)), ((), ())), preferred_element_type=jnp.float32
    )


def _fused_kernel(x_ref, h_ref, h8_ref, bih_ref, bhh_ref, bout_ref,
                  wir_ref, wiz_ref, win_ref, whr_ref, whz_ref, whn_ref,
                  wo_ref, out_ref, hnew_ref, lg_ref):
    i = pl.program_id(0)
    x = x_ref[...]
    h = h_ref[...]

    i_r = _mv(x, wir_ref[...]) + bih_ref[pl.ds(i, 1), :]
    i_z = _mv(x, wiz_ref[...]) + bih_ref[pl.ds(i + NBLK, 1), :]
    i_n = _mv(x, win_ref[...]) + bih_ref[pl.ds(i + 2 * NBLK, 1), :]
    h_r = _mv(h, whr_ref[...]) + bhh_ref[pl.ds(i, 1), :]
    h_z = _mv(h, whz_ref[...]) + bhh_ref[pl.ds(i + NBLK, 1), :]
    h_n = _mv(h, whn_ref[...]) + bhh_ref[pl.ds(i + 2 * NBLK, 1), :]

    r = jax.nn.sigmoid(i_r + h_r)
    z = jax.nn.sigmoid(i_z + h_z)
    n = jnp.tanh(i_n + r * h_n)
    hb = h8_ref[pl.ds(i, 1), :]
    hnew_ref[...] = ((1.0 - z) * n + z * hb).reshape(1, 1, BLK)

    wo = wo_ref[...]
    lx = _mv(x, wo[:, :S])
    lh = _mv(h, wo[:, S:2 * S] + wo[:, 2 * S:3 * S])
    lg_ref[pl.ds(i, 1), :] = lx + lh + bout_ref[pl.ds(i, 1), :]

    @pl.when(i == NBLK - 1)
    def _():
        logits = lg_ref[...]
        m = jnp.max(logits)
        lse = m + jnp.log(jnp.sum(jnp.exp(logits - m)))
        out_ref[...] = (logits - lse).reshape(1, S)


@functools.partial(jax.jit, static_argnames=())
def _run(feature, initial_h, W_ih, W_hh, b_ih, b_hh, W_out, b_out):
    x2 = feature.reshape(1, S)
    h2 = initial_h.reshape(1, S)
    h8 = initial_h.reshape(NBLK, BLK)
    bih = b_ih.reshape(3 * NBLK, BLK)
    bhh = b_hh.reshape(3 * NBLK, BLK)
    bout = b_out.reshape(NBLK, BLK)

    full = lambda shape: pl.BlockSpec(shape, lambda i: tuple(0 for _ in shape))
    gate = lambda off: pl.BlockSpec((BLK, S), lambda i, o=off: (i + o, 0))

    out, h_new = pl.pallas_call(
        _fused_kernel,
        grid=(NBLK,),
        in_specs=[
            full((1, S)),
            full((1, S)),
            full((NBLK, BLK)),
            full((3 * NBLK, BLK)),
            full((3 * NBLK, BLK)),
            full((NBLK, BLK)),
            gate(0),
            gate(NBLK),
            gate(2 * NBLK),
            gate(0),
            gate(NBLK),
            gate(2 * NBLK),
            pl.BlockSpec((BLK, 3 * S + 1), lambda i: (i, 0)),
        ],
        out_specs=[
            pl.BlockSpec((1, S), lambda i: (0, 0)),
            pl.BlockSpec((1, 1, BLK), lambda i: (0, 0, i)),
        ],
        out_shape=[
            jax.ShapeDtypeStruct((1, S), jnp.float32),
            jax.ShapeDtypeStruct((1, 1, S), jnp.float32),
        ],
        scratch_shapes=[pltpu.VMEM((NBLK, BLK), jnp.float32)],
        compiler_params=pltpu.CompilerParams(
            dimension_semantics=("arbitrary",),
        ),
    )(x2, h2, h8, bih, bhh, bout,
      W_ih, W_ih, W_ih, W_hh, W_hh, W_hh, W_out)
    return out, h_new


def kernel(feature, time, initial_h, W_ih, W_hh, b_ih, b_hh, W_out, b_out):
    del time
    return _run(feature, initial_h, W_ih, W_hh, b_ih, b_hh, W_out, b_out)
